# Initial kernel scaffold; baseline (speedup 1.0000x reference)
import jax
import jax.numpy as jnp
from jax.experimental import pallas as pl


def kernel(vertices, faces, edges, face_masks, edge_masks, emb_norm, emb_area, emb_vertex, emb_angle, W_proj, b_proj):
    raise NotImplementedError("write your pallas kernel here")


# speedup vs baseline: 33.7189x; 33.7189x over previous
import functools

import jax
import jax.numpy as jnp
from jax import lax
from jax.experimental import pallas as pl
from jax.experimental.pallas import tpu as pltpu
from jax.experimental.pallas import tpu_sc as plsc

NUM_BINS = 128
SLOT_PAD = 136
NSLOT = 16
PROWS = NSLOT * SLOT_PAD
_NC, _NS = 2, 16
_NW = _NC * _NS

_PI = 3.141592653589793
_HL = [(1.0, -1.0)] * 3 + [(2.0, 0.0)] + [(0.5, -0.5)] * 9 + [(_PI, 0.0)] * 3


def _proj_body(t_ref, w_ref, p_ref):
    p_ref[...] = jnp.dot(t_ref[...], w_ref[0],
                         preferred_element_type=jnp.float32).astype(jnp.bfloat16)


def _sc_gather_body(vx_hbm, vy_hbm, vz_hbm, fidx0_hbm, fidx1_hbm, fidx2_hbm,
                    out_hbm, vx_v, vy_v, vz_v, idx_v, out_v):
    fidx = (fidx0_hbm, fidx1_hbm, fidx2_hbm)
    vcomp_v = (vx_v, vy_v, vz_v)
    n = fidx0_hbm.shape[0]
    chunk = n // _NW
    wid = lax.axis_index("s") * _NC + lax.axis_index("c")
    base = wid * chunk
    pltpu.sync_copy(vx_hbm, vx_v)
    pltpu.sync_copy(vy_hbm, vy_v)
    pltpu.sync_copy(vz_hbm, vz_v)
    for k in range(3):
        pltpu.sync_copy(fidx[k].at[pl.ds(base, chunk)], idx_v)
        for c in range(3):
            def body(j, carry, k=k, c=c):
                off = pl.multiple_of(j * 16, 16)
                iv = idx_v[pl.ds(off, 16)]
                out_v[pl.ds((k * 3 + c) * chunk + off, 16)] = plsc.load_gather(
                    vcomp_v[c], [iv])
                return carry

            lax.fori_loop(0, chunk // 16, body, 0)
    for r in range(9):
        pltpu.sync_copy(out_v.at[pl.ds(r * chunk, chunk)],
                        out_hbm.at[pl.ds(r * n + base, chunk)])


def _quant(feat, high, low):
    f = jnp.clip(feat, low, high)
    q = ((f - low) / (high - low) * NUM_BINS).astype(jnp.int32)
    return jnp.clip(q, 0, NUM_BINS - 1) + 1


def _acos(x):
    r = 2.0 * lax.atan2(jnp.sqrt(1.0 - x * x), 1.0 + x)
    return jnp.where(x == -1.0, jnp.float32(_PI), r)


def _tc_main_body(coords_ref, p_ref, b_ref, o_ref, *, fb):
    eps = 1e-12
    c = [coords_ref[r:r + 1, :] for r in range(9)]

    def vtx(k):
        return c[3 * k], c[3 * k + 1], c[3 * k + 2]

    v0, v1, v2 = vtx(0), vtx(1), vtx(2)
    e1 = [v0[i] - v2[i] for i in range(3)]
    e2 = [v1[i] - v0[i] for i in range(3)]
    cx = e1[1] * e2[2] - e1[2] * e2[1]
    cy = e1[2] * e2[0] - e1[0] * e2[2]
    cz = e1[0] * e2[1] - e1[1] * e2[0]
    nrm = jnp.sqrt((cx * cx + cy * cy) + cz * cz)
    nsafe = jnp.maximum(nrm, eps)
    feats = [cx / nsafe, cy / nsafe, cz / nsafe, nrm * 0.5]
    feats += c
    verts = (v0, v1, v2)
    for k in range(3):
        a = [verts[(k + 1) % 3][i] - verts[k][i] for i in range(3)]
        b = [verts[(k + 2) % 3][i] - verts[k][i] for i in range(3)]
        na = jnp.maximum(jnp.sqrt((a[0] * a[0] + a[1] * a[1]) + a[2] * a[2]), eps)
        nb = jnp.maximum(jnp.sqrt((b[0] * b[0] + b[1] * b[1]) + b[2] * b[2]), eps)
        cos = ((a[0] / na) * (b[0] / nb) + (a[1] / na) * (b[1] / nb)) \
            + (a[2] / na) * (b[2] / nb)
        cos = jnp.clip(cos, -1.0, 1.0)
        feats.append(_acos(cos))

    iot = lax.broadcasted_iota(jnp.int32, (SLOT_PAD, fb), 0)
    pieces = []
    for f, (hi, lo) in zip(feats, _HL):
        q = _quant(f, hi, lo)
        pieces.append((iot == q).astype(jnp.bfloat16))
    mht = jnp.concatenate(pieces, axis=0)
    x = lax.dot_general(mht, p_ref[...], (((0,), (0,)), ((), ())),
                        preferred_element_type=jnp.float32)
    x = x + b_ref[0:1, :]
    xc = x * jnp.float32(0.7071067690849304)
    o_ref[...] = 0.5 * x * (1.0 + lax.erf(xc))


def kernel(vertices, faces, edges, face_masks, edge_masks,
           emb_norm, emb_area, emb_vertex, emb_angle, W_proj, b_proj):
    B, NV = vertices.shape[0], vertices.shape[1]
    NF = faces.shape[1]
    N = B * NF
    D = emb_norm.shape[1]
    H = W_proj.shape[1]

    faces_m = jnp.where(face_masks[..., None], faces, 0)
    fidx_t = (faces_m + (jnp.arange(B, dtype=jnp.int32) * NV)[:, None, None]
              ).transpose(2, 0, 1).reshape(3, N)
    fidx0, fidx1, fidx2 = fidx_t[0], fidx_t[1], fidx_t[2]
    vcomp = vertices.transpose(2, 0, 1).reshape(3, B * NV)
    vx, vy, vz = vcomp[0], vcomp[1], vcomp[2]

    def pad_slot(t):
        return jnp.pad(t, ((0, SLOT_PAD - t.shape[0]), (0, 0)))

    t_stack = jnp.concatenate(
        [pad_slot(emb_norm)] * 3 + [pad_slot(emb_area)]
        + [pad_slot(emb_vertex)] * 9 + [pad_slot(emb_angle)] * 3, axis=0)
    w_r = W_proj.reshape(NSLOT, D, H)
    b_pad = jnp.broadcast_to(b_proj[None, :], (8, H))

    p_tab = pl.pallas_call(
        _proj_body,
        grid=(NSLOT,),
        in_specs=[pl.BlockSpec((SLOT_PAD, D), lambda i: (i, 0)),
                  pl.BlockSpec((1, D, H), lambda i: (i, 0, 0))],
        out_specs=pl.BlockSpec((SLOT_PAD, H), lambda i: (i, 0)),
        out_shape=jax.ShapeDtypeStruct((PROWS, H), jnp.bfloat16),
    )(t_stack, w_r)

    coords_flat = pl.kernel(
        _sc_gather_body,
        out_type=jax.ShapeDtypeStruct((16 * N,), jnp.float32),
        mesh=plsc.VectorSubcoreMesh(core_axis_name="c", subcore_axis_name="s"),
        compiler_params=pltpu.CompilerParams(needs_layout_passes=False),
        scratch_types=[
            pltpu.VMEM((B * NV,), jnp.float32),
            pltpu.VMEM((B * NV,), jnp.float32),
            pltpu.VMEM((B * NV,), jnp.float32),
            pltpu.VMEM((N // _NW,), jnp.int32),
            pltpu.VMEM((9 * (N // _NW),), jnp.float32),
        ],
    )(vx, vy, vz, fidx0, fidx1, fidx2)
    coords = coords_flat.reshape(16, N)

    FB = 512
    out = pl.pallas_call(
        functools.partial(_tc_main_body, fb=FB),
        grid=(N // FB,),
        in_specs=[pl.BlockSpec((16, FB), lambda i: (0, i)),
                  pl.BlockSpec((PROWS, H), lambda i: (0, 0)),
                  pl.BlockSpec((8, H), lambda i: (0, 0))],
        out_specs=pl.BlockSpec((FB, H), lambda i: (i, 0)),
        out_shape=jax.ShapeDtypeStruct((N, H), jnp.float32),
    )(coords, p_tab, b_pad)
    return out.reshape(B, NF, H)
